# baseline (speedup 1.0000x reference)
import functools

import jax
import jax.numpy as jnp
from jax import lax
from jax.experimental import pallas as pl
from jax.experimental.pallas import tpu as pltpu
from jax.experimental.pallas import tpu_sc as plsc

N = 10000
NPAD = 10240
E = 320000
NC, NS = 2, 16
CH = 128
NCHUNK = 80
EPAD = NC * NS * NCHUNK * CH
ROWS_PER_TILE = NPAD // NS
R = 1024
GRID = NPAD // R

_MESH = plsc.VectorSubcoreMesh(
    core_axis_name="c", subcore_axis_name="s", num_cores=NC, num_subcores=NS)
_SC_PARAMS = pltpu.CompilerParams(use_tc_tiling_on_sc=False)


def _deg_body(dst_hbm, ones_hbm, zeros_hbm, out_hbm, dst_v, ones_v, stage, acc):
    c = lax.axis_index("c")
    s = lax.axis_index("s")
    base = s * ROWS_PER_TILE
    pltpu.sync_copy(zeros_hbm, stage)
    pltpu.sync_copy(stage, acc.at[pl.ds(base, ROWS_PER_TILE)])
    pltpu.sync_copy(
        dst_hbm.at[pl.ds((c * NS + s) * NCHUNK, NCHUNK)], dst_v)
    pltpu.sync_copy(ones_hbm, ones_v)
    plsc.subcore_barrier()

    def jb(j, carry):
        pltpu.sync_copy(ones_v, acc.at[dst_v.at[j]], add=True)
        return carry

    lax.fori_loop(0, NCHUNK, jb, 0)
    plsc.subcore_barrier()
    pltpu.sync_copy(acc.at[pl.ds(base, ROWS_PER_TILE)], stage)
    pltpu.sync_copy(stage, out_hbm.at[c, pl.ds(base, ROWS_PER_TILE)])


_deg_kernel = functools.partial(
    pl.kernel, _deg_body,
    out_type=jax.ShapeDtypeStruct((NC, NPAD, 8), jnp.float32),
    mesh=_MESH,
    scratch_types=[
        pltpu.VMEM((NCHUNK, CH), jnp.int32),
        pltpu.VMEM((CH, 8), jnp.float32),
        pltpu.VMEM((ROWS_PER_TILE, 8), jnp.float32),
        pltpu.VMEM_SHARED((NPAD, 8), jnp.float32),
    ],
    compiler_params=_SC_PARAMS,
)()


NBUF = 8
AHEAD = 4
NCHUNK2 = EPAD // (NS * CH)


def _agg_body(g_hbm, src_hbm, dst_hbm, drain_hbm, zeros_hbm, out_hbm,
              src_v, dst_v, drain_v, rows, acc, sg, ss):
    c = lax.axis_index("c")
    s = lax.axis_index("s")
    base = s * ROWS_PER_TILE
    ndrain = ROWS_PER_TILE // CH
    pltpu.sync_copy(zeros_hbm, rows.at[0])
    for k in range(ndrain):
        pltpu.sync_copy(rows.at[0], acc.at[pl.ds(base + k * CH, CH)])
    pltpu.sync_copy(src_hbm.at[pl.ds(s * NCHUNK2, NCHUNK2)], src_v)
    pltpu.sync_copy(dst_hbm.at[pl.ds(s * NCHUNK2, NCHUNK2)], dst_v)
    pltpu.sync_copy(drain_hbm.at[c, s], drain_v)

    def _scale(r, carry):
        for k in range(CH // 16):
            v = src_v[r, pl.ds(k * 16, 16)]
            src_v[r, pl.ds(k * 16, 16)] = v * 2 + c
        return carry

    lax.fori_loop(0, NCHUNK2, _scale, 0)
    plsc.subcore_barrier()

    def _gather(j, b):
        pltpu.async_copy(g_hbm.at[src_v.at[j]], rows.at[b], sg.at[b])

    def _gather_wait(j, b):
        pltpu.make_async_copy(g_hbm.at[src_v.at[j]], rows.at[b], sg.at[b]).wait()

    def _scatter(j, b):
        pltpu.async_copy(rows.at[b], acc.at[dst_v.at[j]], ss.at[b], add=True)

    def _scatter_wait(j, b):
        pltpu.make_async_copy(rows.at[b], acc.at[dst_v.at[j]], ss.at[b]).wait()

    for b in range(AHEAD):
        _gather(b, b)

    def jb(jh, carry):
        for b in range(NBUF):
            j = jh * NBUF + b
            bn = (b + AHEAD) % NBUF

            @pl.when(jnp.logical_and(j + AHEAD < NCHUNK2, j >= AHEAD))
            def _w():
                _scatter_wait(j - AHEAD, bn)

            @pl.when(j + AHEAD < NCHUNK2)
            def _g():
                _gather(j + AHEAD, bn)

            _gather_wait(j, b)
            _scatter(j, b)
        return carry

    lax.fori_loop(0, NCHUNK2 // NBUF, jb, 0)
    for b in range(NBUF):
        _scatter_wait(NCHUNK2 - NBUF + b, b)
    plsc.subcore_barrier()
    for k in range(ndrain):
        pltpu.async_copy(acc.at[pl.ds(base + k * CH, CH)], rows.at[k], sg.at[k])
    for k in range(ndrain):
        pltpu.make_async_copy(
            acc.at[pl.ds(base + k * CH, CH)], rows.at[k], sg.at[k]).wait()
        pltpu.async_copy(rows.at[k], out_hbm.at[drain_v.at[k]], ss.at[k])
    for k in range(ndrain):
        pltpu.make_async_copy(
            rows.at[k], out_hbm.at[drain_v.at[k]], ss.at[k]).wait()


def _make_agg(d):
    h = d // 2
    return functools.partial(
        pl.kernel, _agg_body,
        out_type=jax.ShapeDtypeStruct((2 * NPAD, h), jnp.float32),
        mesh=_MESH,
        scratch_types=[
            pltpu.VMEM((NCHUNK2, CH), jnp.int32),
            pltpu.VMEM((NCHUNK2, CH), jnp.int32),
            pltpu.VMEM((ROWS_PER_TILE // CH, CH), jnp.int32),
            pltpu.VMEM((NBUF, CH, h), jnp.float32),
            pltpu.VMEM_SHARED((NPAD, h), jnp.float32),
            pltpu.SemaphoreType.DMA((NBUF,)),
            pltpu.SemaphoreType.DMA((NBUF,)),
        ],
        compiler_params=_SC_PARAMS,
    )()


_agg64 = _make_agg(64)
_agg32 = _make_agg(32)


def _tc1_body(pr, xr, wr, g_ref, dis_ref):
    deg = pr[0, :, :1] + pr[1, :, :1] + 1.0
    dis = lax.rsqrt(deg)
    h = jnp.dot(xr[...], wr[...], preferred_element_type=jnp.float32)
    g_ref[...] = h * dis
    dis_ref[...] = jnp.broadcast_to(dis, dis_ref.shape)


def _tc_mid_body(ar, gp, dis, br, wr, g_ref):
    d = dis[:, :1]
    f = jnp.maximum(d * (ar[...] + gp[...]) + br[...], 0.0)
    h = jnp.dot(f, wr[...], preferred_element_type=jnp.float32)
    g_ref[...] = h * d


def _tc_fin_body(ar, gp, dis, br, wcr, bcr, out_ref):
    f = jnp.maximum(
        dis[:, :1] * (ar[...] + gp[...]) + br[...], 0.0)
    logits = jnp.dot(f, wcr[...], preferred_element_type=jnp.float32) + bcr[...]
    col = lax.broadcasted_iota(jnp.int32, logits.shape, 1)
    z = jnp.where(col < 3, logits, -jnp.inf)
    m = jnp.max(z, axis=1, keepdims=True)
    e = jnp.exp(z - m)
    out_ref[...] = e / jnp.sum(e, axis=1, keepdims=True)


def _rows(width):
    return pl.BlockSpec((R, width), lambda i: (i, 0))


def _whole(shape):
    return pl.BlockSpec(shape, lambda i: (0,) * len(shape))


def _prows(width):
    return pl.BlockSpec((2, R, width), lambda i: (0, i, 0))


def _tc1(degp, x_pad, w1):
    return pl.pallas_call(
        _tc1_body,
        grid=(GRID,),
        in_specs=[_prows(8), _rows(128), _whole((128, 64))],
        out_specs=[_rows(64), _rows(8)],
        out_shape=[
            jax.ShapeDtypeStruct((NPAD, 64), jnp.float32),
            jax.ShapeDtypeStruct((NPAD, 8), jnp.float32),
        ],
    )(degp, x_pad, w1)


def _tc_mid(a, gp, dis, b, w, d_in, d_out):
    return pl.pallas_call(
        _tc_mid_body,
        grid=(GRID,),
        in_specs=[_rows(d_in), _rows(d_in), _rows(8),
                  _whole((1, d_in)), _whole((d_in, d_out))],
        out_specs=_rows(d_out),
        out_shape=jax.ShapeDtypeStruct((NPAD, d_out), jnp.float32),
    )(a, gp, dis, b.reshape(1, d_in), w)


def _tc_fin(a, gp, dis, b3, wc_pad, bc_pad):
    return pl.pallas_call(
        _tc_fin_body,
        grid=(GRID,),
        in_specs=[_rows(32), _rows(32), _rows(8),
                  _whole((1, 32)), _whole((32, 128)), _whole((1, 128))],
        out_specs=_rows(128),
        out_shape=jax.ShapeDtypeStruct((NPAD, 128), jnp.float32),
    )(a, gp, dis, b3.reshape(1, 32), wc_pad, bc_pad)


def kernel(x, edge_index, W1, b1, W2, b2, W3, b3, Wc, bc):
    ei = edge_index.astype(jnp.int32)
    pad = N + jnp.arange(EPAD - E, dtype=jnp.int32) % (NPAD - N)
    src_r = jnp.concatenate([ei[0], pad]).reshape(NS * NCHUNK2, CH)
    dst_r = jnp.concatenate([ei[1], pad]).reshape(NS * NCHUNK2, CH)
    drain = (2 * jnp.arange(NPAD, dtype=jnp.int32)).reshape(1, NS, -1, CH) \
        + jnp.arange(2, dtype=jnp.int32).reshape(2, 1, 1, 1)
    x_pad = jnp.pad(x, ((0, NPAD - N), (0, 0)))

    ones8 = jnp.ones((CH, 8), jnp.float32)
    zeros8 = jnp.zeros((ROWS_PER_TILE, 8), jnp.float32)
    zeros32 = jnp.zeros((CH, 32), jnp.float32)
    zeros16 = jnp.zeros((CH, 16), jnp.float32)
    wc_pad = jnp.zeros((32, 128), jnp.float32).at[:, :3].set(Wc)
    bc_pad = jnp.zeros((1, 128), jnp.float32).at[0, :3].set(bc)

    degp = _deg_kernel(dst_r, ones8, zeros8)
    g1, dis = _tc1(degp, x_pad, W1)

    a1 = _agg64(g1.reshape(2 * NPAD, 32), src_r, dst_r, drain,
                zeros32).reshape(NPAD, 64)
    g2 = _tc_mid(a1, g1, dis, b1, W2, 64, 64)

    a2 = _agg64(g2.reshape(2 * NPAD, 32), src_r, dst_r, drain,
                zeros32).reshape(NPAD, 64)
    g3 = _tc_mid(a2, g2, dis, b2, W3, 64, 32)

    a3 = _agg32(g3.reshape(2 * NPAD, 16), src_r, dst_r, drain,
                zeros16).reshape(NPAD, 32)
    probs = _tc_fin(a3, g3, dis, b3, wc_pad, bc_pad)
    return probs[:N, :3]


# speedup vs baseline: 1.0161x; 1.0161x over previous
import functools

import jax
import jax.numpy as jnp
from jax import lax
from jax.experimental import pallas as pl
from jax.experimental.pallas import tpu as pltpu
from jax.experimental.pallas import tpu_sc as plsc

N = 10000
NPAD = 10240
E = 320000
NC, NS = 2, 16
CH = 128
NCHUNK = 80
EPAD = NC * NS * NCHUNK * CH
ROWS_PER_TILE = NPAD // NS
R = 1024
GRID = NPAD // R

_MESH = plsc.VectorSubcoreMesh(
    core_axis_name="c", subcore_axis_name="s", num_cores=NC, num_subcores=NS)
_SC_PARAMS = pltpu.CompilerParams(use_tc_tiling_on_sc=False)


def _deg_body(dst_hbm, ones_hbm, zeros_hbm, out_hbm, dst_v, ones_v, stage, acc):
    c = lax.axis_index("c")
    s = lax.axis_index("s")
    base = s * ROWS_PER_TILE
    pltpu.sync_copy(zeros_hbm, stage)
    pltpu.sync_copy(stage, acc.at[pl.ds(base, ROWS_PER_TILE)])
    pltpu.sync_copy(
        dst_hbm.at[pl.ds((c * NS + s) * NCHUNK, NCHUNK)], dst_v)
    pltpu.sync_copy(ones_hbm, ones_v)
    plsc.subcore_barrier()

    def jb(j, carry):
        pltpu.sync_copy(ones_v, acc.at[dst_v.at[j]], add=True)
        return carry

    lax.fori_loop(0, NCHUNK, jb, 0)
    plsc.subcore_barrier()
    pltpu.sync_copy(acc.at[pl.ds(base, ROWS_PER_TILE)], stage)
    pltpu.sync_copy(stage, out_hbm.at[c, pl.ds(base, ROWS_PER_TILE)])


_deg_kernel = functools.partial(
    pl.kernel, _deg_body,
    out_type=jax.ShapeDtypeStruct((NC, NPAD, 8), jnp.float32),
    mesh=_MESH,
    scratch_types=[
        pltpu.VMEM((NCHUNK, CH), jnp.int32),
        pltpu.VMEM((CH, 8), jnp.float32),
        pltpu.VMEM((ROWS_PER_TILE, 8), jnp.float32),
        pltpu.VMEM_SHARED((NPAD, 8), jnp.float32),
    ],
    compiler_params=_SC_PARAMS,
)()


NBUF = 10
AHEAD = 5
NCHUNK2 = EPAD // (NS * CH)


def _agg_body(g_hbm, src_hbm, dst_hbm, drain_hbm, zeros_hbm, out_hbm,
              src_v, dst_v, drain_v, rows, acc, sg, ss):
    c = lax.axis_index("c")
    s = lax.axis_index("s")
    base = s * ROWS_PER_TILE
    ndrain = ROWS_PER_TILE // CH
    pltpu.sync_copy(zeros_hbm, rows.at[0])
    for k in range(ndrain):
        pltpu.sync_copy(rows.at[0], acc.at[pl.ds(base + k * CH, CH)])
    pltpu.sync_copy(src_hbm.at[pl.ds(s * NCHUNK2, NCHUNK2)], src_v)
    pltpu.sync_copy(dst_hbm.at[pl.ds(s * NCHUNK2, NCHUNK2)], dst_v)
    pltpu.sync_copy(drain_hbm.at[c, s], drain_v)

    def _scale(r, carry):
        for k in range(CH // 16):
            v = src_v[r, pl.ds(k * 16, 16)]
            src_v[r, pl.ds(k * 16, 16)] = v * 2 + c
        return carry

    lax.fori_loop(0, NCHUNK2, _scale, 0)
    plsc.subcore_barrier()

    def _gather(j, b):
        pltpu.async_copy(g_hbm.at[src_v.at[j]], rows.at[b], sg.at[b])

    def _gather_wait(j, b):
        pltpu.make_async_copy(g_hbm.at[src_v.at[j]], rows.at[b], sg.at[b]).wait()

    def _scatter(j, b):
        pltpu.async_copy(rows.at[b], acc.at[dst_v.at[j]], ss.at[b], add=True)

    def _scatter_wait(j, b):
        pltpu.make_async_copy(rows.at[b], acc.at[dst_v.at[j]], ss.at[b]).wait()

    for b in range(AHEAD):
        _gather(b, b)

    def jb(jh, carry):
        for b in range(NBUF):
            j = jh * NBUF + b
            bn = (b + AHEAD) % NBUF

            @pl.when(jnp.logical_and(j + AHEAD < NCHUNK2, j >= AHEAD))
            def _w():
                _scatter_wait(j - AHEAD, bn)

            @pl.when(j + AHEAD < NCHUNK2)
            def _g():
                _gather(j + AHEAD, bn)

            _gather_wait(j, b)
            _scatter(j, b)
        return carry

    lax.fori_loop(0, NCHUNK2 // NBUF, jb, 0)
    for b in range(NBUF):
        _scatter_wait(NCHUNK2 - NBUF + b, b)
    plsc.subcore_barrier()
    for k in range(ndrain):
        pltpu.async_copy(acc.at[pl.ds(base + k * CH, CH)], rows.at[k], sg.at[k])
    for k in range(ndrain):
        pltpu.make_async_copy(
            acc.at[pl.ds(base + k * CH, CH)], rows.at[k], sg.at[k]).wait()
        pltpu.async_copy(rows.at[k], out_hbm.at[drain_v.at[k]], ss.at[k])
    for k in range(ndrain):
        pltpu.make_async_copy(
            rows.at[k], out_hbm.at[drain_v.at[k]], ss.at[k]).wait()


def _make_agg(d):
    h = d // 2
    return functools.partial(
        pl.kernel, _agg_body,
        out_type=jax.ShapeDtypeStruct((2 * NPAD, h), jnp.float32),
        mesh=_MESH,
        scratch_types=[
            pltpu.VMEM((NCHUNK2, CH), jnp.int32),
            pltpu.VMEM((NCHUNK2, CH), jnp.int32),
            pltpu.VMEM((ROWS_PER_TILE // CH, CH), jnp.int32),
            pltpu.VMEM((NBUF, CH, h), jnp.float32),
            pltpu.VMEM_SHARED((NPAD, h), jnp.float32),
            pltpu.SemaphoreType.DMA((NBUF,)),
            pltpu.SemaphoreType.DMA((NBUF,)),
        ],
        compiler_params=_SC_PARAMS,
    )()


_agg64 = _make_agg(64)
_agg32 = _make_agg(32)


def _tc1_body(pr, xr, wr, g_ref, dis_ref):
    deg = pr[0, :, :1] + pr[1, :, :1] + 1.0
    dis = lax.rsqrt(deg)
    h = jnp.dot(xr[...], wr[...], preferred_element_type=jnp.float32)
    g_ref[...] = h * dis
    dis_ref[...] = jnp.broadcast_to(dis, dis_ref.shape)


def _tc_mid_body(ar, gp, dis, br, wr, g_ref):
    d = dis[:, :1]
    f = jnp.maximum(d * (ar[...] + gp[...]) + br[...], 0.0)
    h = jnp.dot(f, wr[...], preferred_element_type=jnp.float32)
    g_ref[...] = h * d


def _tc_fin_body(ar, gp, dis, br, wcr, bcr, out_ref):
    f = jnp.maximum(
        dis[:, :1] * (ar[...] + gp[...]) + br[...], 0.0)
    logits = jnp.dot(f, wcr[...], preferred_element_type=jnp.float32) + bcr[...]
    col = lax.broadcasted_iota(jnp.int32, logits.shape, 1)
    z = jnp.where(col < 3, logits, -jnp.inf)
    m = jnp.max(z, axis=1, keepdims=True)
    e = jnp.exp(z - m)
    p = e / jnp.sum(e, axis=1, keepdims=True)
    out_ref[...] = p[:, :3]


def _rows(width):
    return pl.BlockSpec((R, width), lambda i: (i, 0))


def _whole(shape):
    return pl.BlockSpec(shape, lambda i: (0,) * len(shape))


def _prows(width):
    return pl.BlockSpec((2, R, width), lambda i: (0, i, 0))


def _tc1(degp, x_pad, w1):
    return pl.pallas_call(
        _tc1_body,
        grid=(GRID,),
        in_specs=[_prows(8), _rows(128), _whole((128, 64))],
        out_specs=[_rows(64), _rows(8)],
        out_shape=[
            jax.ShapeDtypeStruct((NPAD, 64), jnp.float32),
            jax.ShapeDtypeStruct((NPAD, 8), jnp.float32),
        ],
    )(degp, x_pad, w1)


def _tc_mid(a, gp, dis, b, w, d_in, d_out):
    return pl.pallas_call(
        _tc_mid_body,
        grid=(GRID,),
        in_specs=[_rows(d_in), _rows(d_in), _rows(8),
                  _whole((1, d_in)), _whole((d_in, d_out))],
        out_specs=_rows(d_out),
        out_shape=jax.ShapeDtypeStruct((NPAD, d_out), jnp.float32),
    )(a, gp, dis, b.reshape(1, d_in), w)


RF = N // GRID


def _tc_fin(a, gp, dis, b3, wc_pad, bc_pad):
    return pl.pallas_call(
        _tc_fin_body,
        grid=(GRID,),
        in_specs=[pl.BlockSpec((RF, 32), lambda i: (i, 0)),
                  pl.BlockSpec((RF, 32), lambda i: (i, 0)),
                  pl.BlockSpec((RF, 8), lambda i: (i, 0)),
                  _whole((1, 32)), _whole((32, 128)), _whole((1, 128))],
        out_specs=pl.BlockSpec((RF, 3), lambda i: (i, 0)),
        out_shape=jax.ShapeDtypeStruct((N, 3), jnp.float32),
    )(a, gp, dis, b3.reshape(1, 32), wc_pad, bc_pad)


def kernel(x, edge_index, W1, b1, W2, b2, W3, b3, Wc, bc):
    ei = edge_index.astype(jnp.int32)
    pad = N + jnp.arange(EPAD - E, dtype=jnp.int32) % (NPAD - N)
    src_r = jnp.concatenate([ei[0], pad]).reshape(NS * NCHUNK2, CH)
    dst_r = jnp.concatenate([ei[1], pad]).reshape(NS * NCHUNK2, CH)
    drain = (2 * jnp.arange(NPAD, dtype=jnp.int32)).reshape(1, NS, -1, CH) \
        + jnp.arange(2, dtype=jnp.int32).reshape(2, 1, 1, 1)
    x_pad = jnp.pad(x, ((0, NPAD - N), (0, 0)))

    ones8 = jnp.ones((CH, 8), jnp.float32)
    zeros8 = jnp.zeros((ROWS_PER_TILE, 8), jnp.float32)
    zeros32 = jnp.zeros((CH, 32), jnp.float32)
    zeros16 = jnp.zeros((CH, 16), jnp.float32)
    wc_pad = jnp.zeros((32, 128), jnp.float32).at[:, :3].set(Wc)
    bc_pad = jnp.zeros((1, 128), jnp.float32).at[0, :3].set(bc)

    degp = _deg_kernel(dst_r, ones8, zeros8)
    g1, dis = _tc1(degp, x_pad, W1)

    a1 = _agg64(g1.reshape(2 * NPAD, 32), src_r, dst_r, drain,
                zeros32).reshape(NPAD, 64)
    g2 = _tc_mid(a1, g1, dis, b1, W2, 64, 64)

    a2 = _agg64(g2.reshape(2 * NPAD, 32), src_r, dst_r, drain,
                zeros32).reshape(NPAD, 64)
    g3 = _tc_mid(a2, g2, dis, b2, W3, 64, 32)

    a3 = _agg32(g3.reshape(2 * NPAD, 16), src_r, dst_r, drain,
                zeros16).reshape(NPAD, 32)
    return _tc_fin(a3, g3, dis, b3, wc_pad, bc_pad)
